# baseline (speedup 1.0000x reference)
import functools

import jax
import jax.numpy as jnp
from jax import lax
from jax.experimental import pallas as pl
from jax.experimental.pallas import tpu as pltpu
from jax.experimental.pallas import tpu_sc as plsc

N_NODES = 10000
D = 128
N_EDGES = 320000

NC = 2
NS = 16
NW = NC * NS
LANES = 16

K = 128
C = 80
E_W = C * K
E_PAD = NW * E_W

ROWS_PER_TILE = 624
TAIL_ROWS = N_NODES - NS * ROWS_PER_TILE
TAIL_OFF = NS * ROWS_PER_TILE


def _sc_aggregate(x, meta, wgt, zeros):
    mesh = plsc.VectorSubcoreMesh(
        core_axis_name="c", subcore_axis_name="s", num_cores=NC, num_subcores=NS
    )

    @functools.partial(
        pl.kernel,
        out_type=jax.ShapeDtypeStruct((NC, N_NODES, D), jnp.float32),
        mesh=mesh,
        scratch_types=[
            pltpu.VMEM_SHARED((N_NODES, D), jnp.float32),
            pltpu.VMEM((2, 2, K), jnp.int32),
            pltpu.VMEM((2, 1, K), jnp.float32),
            pltpu.VMEM((2, K, D), jnp.float32),
            pltpu.SemaphoreType.DMA((2,)),
            pltpu.SemaphoreType.DMA((2,)),
        ],
    )
    def agg(x_hbm, meta_hbm, wgt_hbm, zeros_hbm, part_hbm,
            acc, meta_v, w_v, rows_v, gsem, ssem):
        cid = lax.axis_index("c")
        sid = lax.axis_index("s")
        wid = sid * NC + cid

        pltpu.sync_copy(zeros_hbm.at[pl.ds(sid * ROWS_PER_TILE, ROWS_PER_TILE)],
                        acc.at[pl.ds(sid * ROWS_PER_TILE, ROWS_PER_TILE)])

        @pl.when(sid == 0)
        def _():
            pltpu.sync_copy(zeros_hbm.at[pl.ds(TAIL_OFF, TAIL_ROWS)],
                            acc.at[pl.ds(TAIL_OFF, TAIL_ROWS)])

        plsc.subcore_barrier()

        def fetch_meta(c, b):
            pltpu.sync_copy(meta_hbm.at[wid, c], meta_v.at[b])
            pltpu.sync_copy(wgt_hbm.at[wid, c], w_v.at[b])

        def start_gather(b):
            pltpu.async_copy(x_hbm.at[meta_v.at[b, 0]], rows_v.at[b],
                             gsem.at[b])

        def scale_rows(b):
            def mul_body(eb, carry):
                wvec = w_v[b, 0, pl.ds(eb * LANES, LANES)]
                for j in range(LANES):
                    wb = wvec[j]
                    e = eb * LANES + j
                    for g in range(D // LANES):
                        sl = pl.ds(g * LANES, LANES)
                        rows_v[b, e, sl] = rows_v[b, e, sl] * wb
                return carry
            lax.fori_loop(0, K // LANES, mul_body, 0)

        def start_scatter(b):
            pltpu.async_copy(rows_v.at[b], acc.at[meta_v.at[b, 1]],
                             ssem.at[b], add=True)

        def wait_gather(b):
            pltpu.make_async_copy(x_hbm.at[meta_v.at[b, 0]], rows_v.at[b],
                                  gsem.at[b]).wait()

        def wait_scatter(b):
            pltpu.make_async_copy(rows_v.at[b], acc.at[meta_v.at[b, 1]],
                                  ssem.at[b]).wait()

        fetch_meta(0, 0)
        start_gather(0)
        fetch_meta(1, 1)
        start_gather(1)
        wait_gather(0)
        scale_rows(0)
        start_scatter(0)

        def pair_body(t, carry):
            c0 = 2 * t
            wait_scatter(0)
            fetch_meta(c0, 0)
            start_gather(0)
            wait_gather(1)
            scale_rows(1)
            start_scatter(1)
            wait_scatter(1)
            fetch_meta(c0 + 1, 1)
            start_gather(1)
            wait_gather(0)
            scale_rows(0)
            start_scatter(0)
            return carry

        lax.fori_loop(1, C // 2, pair_body, 0)

        wait_gather(1)
        scale_rows(1)
        start_scatter(1)
        wait_scatter(0)
        wait_scatter(1)

        plsc.subcore_barrier()
        pltpu.sync_copy(acc.at[pl.ds(sid * ROWS_PER_TILE, ROWS_PER_TILE)],
                        part_hbm.at[cid, pl.ds(sid * ROWS_PER_TILE, ROWS_PER_TILE)])

        @pl.when(sid == 0)
        def _():
            pltpu.sync_copy(acc.at[pl.ds(TAIL_OFF, TAIL_ROWS)],
                            part_hbm.at[cid, pl.ds(TAIL_OFF, TAIL_ROWS)])

    return agg(x, meta, wgt, zeros)


def _tc_finish(parts, W):
    R = 1000

    def body(p_ref, w_ref, o_ref):
        p = p_ref[0] + p_ref[1]
        y = jnp.dot(p, w_ref[...], preferred_element_type=jnp.float32)
        o_ref[...] = jnp.maximum(y, 0.0)

    return pl.pallas_call(
        body,
        grid=(N_NODES // R,),
        in_specs=[
            pl.BlockSpec((NC, R, D), lambda i: (0, i, 0)),
            pl.BlockSpec((D, D), lambda i: (0, 0)),
        ],
        out_specs=pl.BlockSpec((R, D), lambda i: (i, 0)),
        out_shape=jax.ShapeDtypeStruct((N_NODES, D), jnp.float32),
    )(parts, W)


def kernel(x, edge_index, edge_weight, W):
    pad = E_PAD - N_EDGES
    ei = jnp.concatenate(
        [edge_index, jnp.zeros((2, pad), edge_index.dtype)], axis=1)
    w = jnp.concatenate([edge_weight, jnp.zeros((pad,), edge_weight.dtype)])
    meta = jnp.stack(
        [ei[1].reshape(NW, C, K), ei[0].reshape(NW, C, K)], axis=2)
    wgt = w.reshape(NW, C, 1, K)
    zeros = jnp.zeros((N_NODES, D), jnp.float32)
    parts = _sc_aggregate(x, meta, wgt, zeros)
    return _tc_finish(parts, W)


# speedup vs baseline: 1.0223x; 1.0223x over previous
import functools

import jax
import jax.numpy as jnp
from jax import lax
from jax.experimental import pallas as pl
from jax.experimental.pallas import tpu as pltpu
from jax.experimental.pallas import tpu_sc as plsc

N_NODES = 10000
D = 128
DH = D // 2
N_EDGES = 320000

NC = 2
NS = 16
LANES = 16

K = 128
C = 162
E_W = C * K
E_PAD = NS * E_W

ROWS_PER_TILE = 624
TAIL_ROWS = N_NODES - NS * ROWS_PER_TILE
TAIL_OFF = NS * ROWS_PER_TILE

NB = 3


def _sc_aggregate(xcat, src2, dst2, wgt, zeros):
    mesh = plsc.VectorSubcoreMesh(
        core_axis_name="c", subcore_axis_name="s", num_cores=NC, num_subcores=NS
    )

    @functools.partial(
        pl.kernel,
        out_type=jax.ShapeDtypeStruct((NC, N_NODES, DH), jnp.float32),
        mesh=mesh,
        scratch_types=[
            pltpu.VMEM_SHARED((N_NODES, DH), jnp.float32),
            pltpu.VMEM((C, K), jnp.int32),
            pltpu.VMEM((C, K), jnp.int32),
            pltpu.VMEM((E_W,), jnp.float32),
            pltpu.VMEM((NB, K, DH), jnp.float32),
            pltpu.SemaphoreType.DMA((NB,)),
            pltpu.SemaphoreType.DMA((NB,)),
        ],
        compiler_params=pltpu.CompilerParams(use_tc_tiling_on_sc=False),
    )
    def agg(x_hbm, src_hbm, dst_hbm, w_hbm, zeros_hbm, part_hbm,
            acc, src_v, dst_v, w_v, rows_v, gsem, ssem):
        cid = lax.axis_index("c")
        sid = lax.axis_index("s")

        pltpu.sync_copy(zeros_hbm.at[pl.ds(sid * ROWS_PER_TILE, ROWS_PER_TILE)],
                        acc.at[pl.ds(sid * ROWS_PER_TILE, ROWS_PER_TILE)])

        @pl.when(sid == 0)
        def _():
            pltpu.sync_copy(zeros_hbm.at[pl.ds(TAIL_OFF, TAIL_ROWS)],
                            acc.at[pl.ds(TAIL_OFF, TAIL_ROWS)])

        pltpu.sync_copy(src_hbm.at[cid, sid], src_v)
        pltpu.sync_copy(dst_hbm.at[sid], dst_v)
        pltpu.sync_copy(w_hbm.at[sid], w_v)
        plsc.subcore_barrier()

        def start_gather(c, b):
            pltpu.async_copy(x_hbm.at[src_v.at[c]], rows_v.at[b], gsem.at[b])

        def wait_gather(c, b):
            pltpu.make_async_copy(x_hbm.at[src_v.at[c]], rows_v.at[b],
                                  gsem.at[b]).wait()

        def start_scatter(c, b):
            pltpu.async_copy(rows_v.at[b], acc.at[dst_v.at[c]], ssem.at[b],
                             add=True)

        def wait_scatter(c, b):
            pltpu.make_async_copy(rows_v.at[b], acc.at[dst_v.at[c]],
                                  ssem.at[b]).wait()

        def scale(c, b):
            def mul_body(eb, carry):
                wvec = w_v[pl.ds(c * K + eb * LANES, LANES)]
                for j in range(LANES):
                    wb = wvec[j]
                    e = eb * LANES + j
                    for g in range(DH // LANES):
                        sl = pl.ds(g * LANES, LANES)
                        rows_v[b, e, sl] = rows_v[b, e, sl] * wb
                return carry
            lax.fori_loop(0, K // LANES, mul_body, 0)

        def step(i, b, first_two, last):
            wait_gather(i, b)
            if not first_two:
                wait_scatter(i - 2, (b + 1) % NB)
            if not last:
                start_gather(i + 1, (b + 1) % NB)
            scale(i, b)
            start_scatter(i, b)

        start_gather(0, 0)
        step(0, 0, True, False)
        step(1, 1, True, False)
        step(2, 2, False, False)

        def main_body(t, carry):
            i = 3 * t
            step(i, 0, False, False)
            step(i + 1, 1, False, False)
            step(i + 2, 2, False, False)
            return carry

        lax.fori_loop(1, C // 3 - 1, main_body, 0)

        step(C - 3, 0, False, False)
        step(C - 2, 1, False, False)
        step(C - 1, 2, False, True)
        wait_scatter(C - 2, 1)
        wait_scatter(C - 1, 2)

        plsc.subcore_barrier()
        pltpu.sync_copy(acc.at[pl.ds(sid * ROWS_PER_TILE, ROWS_PER_TILE)],
                        part_hbm.at[cid, pl.ds(sid * ROWS_PER_TILE, ROWS_PER_TILE)])

        @pl.when(sid == 0)
        def _():
            pltpu.sync_copy(acc.at[pl.ds(TAIL_OFF, TAIL_ROWS)],
                            part_hbm.at[cid, pl.ds(TAIL_OFF, TAIL_ROWS)])

    return agg(xcat, src2, dst2, wgt, zeros)


def _tc_finish(parts, W):
    R = 1000

    def body(p_ref, w_ref, o_ref):
        p = jnp.concatenate([p_ref[0], p_ref[1]], axis=-1)
        y = jnp.dot(p, w_ref[...], preferred_element_type=jnp.float32)
        o_ref[...] = jnp.maximum(y, 0.0)

    return pl.pallas_call(
        body,
        grid=(N_NODES // R,),
        in_specs=[
            pl.BlockSpec((NC, R, DH), lambda i: (0, i, 0)),
            pl.BlockSpec((D, D), lambda i: (0, 0)),
        ],
        out_specs=pl.BlockSpec((R, D), lambda i: (i, 0)),
        out_shape=jax.ShapeDtypeStruct((N_NODES, D), jnp.float32),
    )(parts, W)


def kernel(x, edge_index, edge_weight, W):
    pad = E_PAD - N_EDGES
    ei = jnp.concatenate(
        [edge_index, jnp.zeros((2, pad), edge_index.dtype)], axis=1)
    w = jnp.concatenate([edge_weight, jnp.zeros((pad,), edge_weight.dtype)])
    src = ei[1].reshape(NS, C, K)
    dst2 = ei[0].reshape(NS, C, K)
    src2 = jnp.stack([src, src + N_NODES])
    wgt = w.reshape(NS, E_W)
    xcat = jnp.concatenate([x[:, :DH], x[:, DH:]], axis=0)
    zeros = jnp.zeros((N_NODES, DH), jnp.float32)
    parts = _sc_aggregate(xcat, src2, dst2, wgt, zeros)
    return _tc_finish(parts, W)


# speedup vs baseline: 1.7762x; 1.7375x over previous
import functools

import jax
import jax.numpy as jnp
from jax import lax
from jax.experimental import pallas as pl
from jax.experimental.pallas import tpu as pltpu
from jax.experimental.pallas import tpu_sc as plsc

N_NODES = 10000
D = 128
N_EDGES = 320000

NC = 2
NS = 16
NW = NC * NS
LANES = 16

K = 80
C = 126
E_W = C * K
E_PAD = NW * E_W

ROWS_PER_TILE = 624
TAIL_ROWS = N_NODES - NS * ROWS_PER_TILE
TAIL_OFF = NS * ROWS_PER_TILE

NB = 3


def _sc_aggregate(x, src3, dst3, wgt, zeros):
    mesh = plsc.VectorSubcoreMesh(
        core_axis_name="c", subcore_axis_name="s", num_cores=NC, num_subcores=NS
    )

    @functools.partial(
        pl.kernel,
        out_type=jax.ShapeDtypeStruct((NC, N_NODES, D), jnp.float32),
        mesh=mesh,
        scratch_types=[
            pltpu.VMEM_SHARED((N_NODES, D), jnp.float32),
            pltpu.VMEM((C, K), jnp.int32),
            pltpu.VMEM((NB, 1, K), jnp.int32),
            pltpu.VMEM((NB, 1, K), jnp.float32),
            pltpu.VMEM((NB, K, D), jnp.float32),
            pltpu.SemaphoreType.DMA((NB,)),
            pltpu.SemaphoreType.DMA((NB,)),
            pltpu.SemaphoreType.DMA((NB,)),
            pltpu.SemaphoreType.DMA((NB,)),
        ],
        compiler_params=pltpu.CompilerParams(use_tc_tiling_on_sc=False),
    )
    def agg(x_hbm, src_hbm, dst_hbm, w_hbm, zeros_hbm, part_hbm,
            acc, dst_v, src_v, w_v, rows_v, fsem, wsem, gsem, ssem):
        cid = lax.axis_index("c")
        sid = lax.axis_index("s")
        wid = sid * NC + cid

        pltpu.sync_copy(zeros_hbm.at[pl.ds(sid * ROWS_PER_TILE, ROWS_PER_TILE)],
                        acc.at[pl.ds(sid * ROWS_PER_TILE, ROWS_PER_TILE)])

        @pl.when(sid == 0)
        def _():
            pltpu.sync_copy(zeros_hbm.at[pl.ds(TAIL_OFF, TAIL_ROWS)],
                            acc.at[pl.ds(TAIL_OFF, TAIL_ROWS)])

        pltpu.sync_copy(dst_hbm.at[wid], dst_v)
        plsc.subcore_barrier()

        def start_fetch(c, b):
            pltpu.async_copy(src_hbm.at[wid, c], src_v.at[b], fsem.at[b])
            pltpu.async_copy(w_hbm.at[wid, c], w_v.at[b], wsem.at[b])

        def wait_fetch_src(c, b):
            pltpu.make_async_copy(src_hbm.at[wid, c], src_v.at[b],
                                  fsem.at[b]).wait()

        def wait_fetch_w(c, b):
            pltpu.make_async_copy(w_hbm.at[wid, c], w_v.at[b],
                                  wsem.at[b]).wait()

        def start_gather(c, b):
            pltpu.async_copy(x_hbm.at[src_v.at[b, 0]], rows_v.at[b],
                             gsem.at[b])

        def wait_gather(c, b):
            pltpu.make_async_copy(x_hbm.at[src_v.at[b, 0]], rows_v.at[b],
                                  gsem.at[b]).wait()

        def start_scatter(c, b):
            pltpu.async_copy(rows_v.at[b], acc.at[dst_v.at[c]], ssem.at[b],
                             add=True)

        def wait_scatter(c, b):
            pltpu.make_async_copy(rows_v.at[b], acc.at[dst_v.at[c]],
                                  ssem.at[b]).wait()

        def scale(b):
            def mul_body(eb, carry):
                wvec = w_v[b, 0, pl.ds(eb * LANES, LANES)]
                for j in range(LANES):
                    wb = wvec[j]
                    e = eb * LANES + j
                    for g in range(D // LANES):
                        sl = pl.ds(g * LANES, LANES)
                        rows_v[b, e, sl] = rows_v[b, e, sl] * wb
                return carry
            lax.fori_loop(0, K // LANES, mul_body, 0)

        def stepc(i, b, head=False, tail2=False, tail1=False):
            if not tail2:
                start_fetch(i + 2, (b + 2) % NB)
            wait_gather(i, b)
            if not head:
                wait_scatter(i - 2, (b + 1) % NB)
            if not tail1:
                wait_fetch_src(i + 1, (b + 1) % NB)
                start_gather(i + 1, (b + 1) % NB)
            wait_fetch_w(i, b)
            scale(b)
            start_scatter(i, b)

        start_fetch(0, 0)
        start_fetch(1, 1)
        wait_fetch_src(0, 0)
        start_gather(0, 0)
        stepc(0, 0, head=True)
        stepc(1, 1, head=True)

        def main_body(t, carry):
            i = 3 * t + 2
            stepc(i, 2)
            stepc(i + 1, 0)
            stepc(i + 2, 1)
            return carry

        lax.fori_loop(0, (C - 6) // 3, main_body, 0)

        stepc(C - 4, (C - 4) % NB)
        stepc(C - 3, (C - 3) % NB)
        stepc(C - 2, (C - 2) % NB, tail2=True)
        stepc(C - 1, (C - 1) % NB, tail2=True, tail1=True)
        wait_scatter(C - 2, (C - 2) % NB)
        wait_scatter(C - 1, (C - 1) % NB)

        plsc.subcore_barrier()
        pltpu.sync_copy(acc.at[pl.ds(sid * ROWS_PER_TILE, ROWS_PER_TILE)],
                        part_hbm.at[cid, pl.ds(sid * ROWS_PER_TILE, ROWS_PER_TILE)])

        @pl.when(sid == 0)
        def _():
            pltpu.sync_copy(acc.at[pl.ds(TAIL_OFF, TAIL_ROWS)],
                            part_hbm.at[cid, pl.ds(TAIL_OFF, TAIL_ROWS)])

    return agg(x, src3, dst3, wgt, zeros)


def _tc_finish(parts, W):
    R = 1000

    def body(p_ref, w_ref, o_ref):
        p = p_ref[0] + p_ref[1]
        y = jnp.dot(p, w_ref[...], preferred_element_type=jnp.float32)
        o_ref[...] = jnp.maximum(y, 0.0)

    return pl.pallas_call(
        body,
        grid=(N_NODES // R,),
        in_specs=[
            pl.BlockSpec((NC, R, D), lambda i: (0, i, 0)),
            pl.BlockSpec((D, D), lambda i: (0, 0)),
        ],
        out_specs=pl.BlockSpec((R, D), lambda i: (i, 0)),
        out_shape=jax.ShapeDtypeStruct((N_NODES, D), jnp.float32),
    )(parts, W)


def kernel(x, edge_index, edge_weight, W):
    pad = E_PAD - N_EDGES
    ei = jnp.concatenate(
        [edge_index, jnp.zeros((2, pad), edge_index.dtype)], axis=1)
    w = jnp.concatenate([edge_weight, jnp.zeros((pad,), edge_weight.dtype)])
    src3 = ei[1].reshape(NW, C, 1, K)
    dst3 = ei[0].reshape(NW, C, K)
    wgt = w.reshape(NW, C, 1, K)
    zeros = jnp.zeros((N_NODES, D), jnp.float32)
    parts = _sc_aggregate(x, src3, dst3, wgt, zeros)
    return _tc_finish(parts, W)
